# baseline (speedup 1.0000x reference)
import functools

import jax
import jax.numpy as jnp
from jax import lax
from jax.experimental import pallas as pl
from jax.experimental.pallas import tpu as pltpu
from jax.experimental.pallas import tpu_sc as plsc

_N = 100000
_E = 3200000
_D_EDGE = 16
_LANE = 128
_R = _E // _LANE
_OCT = _R // 8
_NC = 2
_NS = 16
_NW = _NC * _NS
_OCT_PER_W = _OCT // _NW
_EXTRA = _OCT - _OCT_PER_W * _NW
_CH = 8
_CH_E = _CH * _LANE
_NPAD = 100096
_NPT = _NPAD // _NS


def _seg_sum_body(ei_hbm, ea_hbm, out0_hbm, out1_hbm, idx_buf, rows_buf, acc):
    cid = lax.axis_index("c")
    sid = lax.axis_index("s")
    wid = cid * _NS + sid

    def _zero_row(i, _):
        rows_buf[i, :] = jnp.zeros((_D_EDGE,), jnp.float32)
        return 0

    lax.fori_loop(0, _CH_E, _zero_row, 0)
    base_n = sid * _NPT
    for k in range(6):
        pltpu.sync_copy(rows_buf, acc.at[pl.ds(base_n + k * _CH_E, _CH_E)])
    rem0 = _NPT - 6 * _CH_E
    pltpu.sync_copy(rows_buf.at[pl.ds(0, rem0)],
                    acc.at[pl.ds(base_n + 6 * _CH_E, rem0)])
    plsc.subcore_barrier()

    base_row = (wid * _OCT_PER_W + jnp.minimum(wid, _EXTRA)) * _CH
    nchunks = _OCT_PER_W + jnp.where(wid < _EXTRA, 1, 0)

    def _chunk(i, _):
        row0 = base_row + i * _CH
        e0 = row0 * _LANE
        pltpu.sync_copy(ei_hbm.at[1, pl.ds(e0, _CH_E)], idx_buf)
        pltpu.sync_copy(ea_hbm.at[pl.ds(e0, _CH_E)], rows_buf)
        for j in range(_CH):
            pltpu.sync_copy(rows_buf.at[pl.ds(j * _LANE, _LANE)],
                            acc.at[idx_buf.at[pl.ds(j * _LANE, _LANE)]],
                            add=True)
        return 0

    lax.fori_loop(0, nchunks, _chunk, 0)
    plsc.subcore_barrier()

    @pl.when(cid == 0)
    def _():
        pltpu.sync_copy(acc.at[pl.ds(base_n, _NPT)],
                        out0_hbm.at[pl.ds(base_n, _NPT)])

    @pl.when(cid == 1)
    def _():
        pltpu.sync_copy(acc.at[pl.ds(base_n, _NPT)],
                        out1_hbm.at[pl.ds(base_n, _NPT)])


_seg_sum = pl.kernel(
    _seg_sum_body,
    out_type=(jax.ShapeDtypeStruct((_NPAD, _D_EDGE), jnp.float32),
              jax.ShapeDtypeStruct((_NPAD, _D_EDGE), jnp.float32)),
    mesh=plsc.VectorSubcoreMesh(core_axis_name="c", subcore_axis_name="s"),
    compiler_params=pltpu.CompilerParams(use_tc_tiling_on_sc=False),
    scratch_types=[
        pltpu.VMEM((_CH_E,), jnp.int32),
        pltpu.VMEM((_CH_E, _D_EDGE), jnp.float32),
        pltpu.VMEM_SHARED((_NPAD, _D_EDGE), jnp.float32),
    ],
)


_BN = 2000


def _mlp_body(x_ref, p0_ref, p1_ref, g_ref, w1x_ref, w1a_ref, w1g_ref,
              b1_ref, w2_ref, b2_ref, w3_ref, b3_ref, w4_ref, b4_ref,
              out_ref):
    agg = p0_ref[...] + p1_ref[...]
    c = (jnp.dot(g_ref[...], w1g_ref[...], preferred_element_type=jnp.float32)
         + b1_ref[...])
    h = (jnp.dot(x_ref[...], w1x_ref[...], preferred_element_type=jnp.float32)
         + jnp.dot(agg, w1a_ref[...], preferred_element_type=jnp.float32)
         + c)
    h = jnp.maximum(h, 0.0)
    h = jnp.dot(h, w2_ref[...], preferred_element_type=jnp.float32) + b2_ref[...]
    h = jnp.maximum(h, 0.0)
    h = jnp.dot(h, w3_ref[...], preferred_element_type=jnp.float32) + b3_ref[...]
    h = jnp.maximum(h, 0.0)
    out_ref[...] = (jnp.dot(h, w4_ref[...], preferred_element_type=jnp.float32)
                    + b4_ref[...])


def _row_block(i):
    return (i, 0)


def _whole(i):
    return (0, 0)


@functools.partial(jax.jit, static_argnums=())
def kernel(x, edge_index, edge_attr, global_attr,
           W1, b1, W2, b2, W3, b3, W4, b4):
    p0, p1 = _seg_sum(edge_index, edge_attr)

    d_node = x.shape[1]
    w1x = W1[:, :d_node].T
    w1a = W1[:, d_node:d_node + _D_EDGE].T
    w1g = W1[:, d_node + _D_EDGE:].T

    grid = (_N // _BN,)
    out = pl.pallas_call(
        _mlp_body,
        grid=grid,
        in_specs=[
            pl.BlockSpec((_BN, d_node), _row_block),
            pl.BlockSpec((_BN, _D_EDGE), _row_block),
            pl.BlockSpec((_BN, _D_EDGE), _row_block),
            pl.BlockSpec(global_attr.shape, _whole),
            pl.BlockSpec(w1x.shape, _whole),
            pl.BlockSpec(w1a.shape, _whole),
            pl.BlockSpec(w1g.shape, _whole),
            pl.BlockSpec((1, b1.shape[0]), _whole),
            pl.BlockSpec(W2.shape, _whole),
            pl.BlockSpec((1, b2.shape[0]), _whole),
            pl.BlockSpec(W3.shape, _whole),
            pl.BlockSpec((1, b3.shape[0]), _whole),
            pl.BlockSpec(W4.shape, _whole),
            pl.BlockSpec((1, b4.shape[0]), _whole),
        ],
        out_specs=pl.BlockSpec((_BN, W4.shape[0]), _row_block),
        out_shape=jax.ShapeDtypeStruct((_N, W4.shape[0]), jnp.float32),
    )(x, p0, p1, global_attr, w1x, w1a, w1g, b1.reshape(1, -1),
      W2.T, b2.reshape(1, -1), W3.T, b3.reshape(1, -1), W4.T,
      b4.reshape(1, -1))
    return out


# speedup vs baseline: 1.0024x; 1.0024x over previous
import functools

import jax
import jax.numpy as jnp
from jax import lax
from jax.experimental import pallas as pl
from jax.experimental.pallas import tpu as pltpu
from jax.experimental.pallas import tpu_sc as plsc

_N = 100000
_E = 3200000
_D_EDGE = 16
_LANE = 128
_R = _E // _LANE
_OCT = _R // 8
_NC = 2
_NS = 16
_NW = _NC * _NS
_OCT_PER_W = _OCT // _NW
_EXTRA = _OCT - _OCT_PER_W * _NW
_CH = 8
_CH_E = _CH * _LANE
_NPAD = 100096
_NPT = _NPAD // _NS


def _extract_body(ei_ref, out_ref):
    out_ref[...] = ei_ref[1, :].reshape(out_ref.shape)


def _extract_col(edge_index):
    nblk = 25
    be = _E // nblk
    br = be // _LANE
    return pl.pallas_call(
        _extract_body,
        grid=(nblk,),
        in_specs=[pl.BlockSpec((2, be), lambda i: (0, i))],
        out_specs=pl.BlockSpec((br, _LANE), lambda i: (i, 0)),
        out_shape=jax.ShapeDtypeStruct((_R, _LANE), jnp.int32),
    )(edge_index)


def _seg_sum_body(col_hbm, ea_hbm, out0_hbm, out1_hbm, idx_buf, rows_buf, acc):
    cid = lax.axis_index("c")
    sid = lax.axis_index("s")
    wid = cid * _NS + sid

    def _zero_row(i, _):
        rows_buf[i, :] = jnp.zeros((_D_EDGE,), jnp.float32)
        return 0

    lax.fori_loop(0, _CH_E, _zero_row, 0)
    base_n = sid * _NPT
    for k in range(6):
        pltpu.sync_copy(rows_buf, acc.at[pl.ds(base_n + k * _CH_E, _CH_E)])
    rem0 = _NPT - 6 * _CH_E
    pltpu.sync_copy(rows_buf.at[pl.ds(0, rem0)],
                    acc.at[pl.ds(base_n + 6 * _CH_E, rem0)])
    plsc.subcore_barrier()

    base_row = (wid * _OCT_PER_W + jnp.minimum(wid, _EXTRA)) * _CH
    nchunks = _OCT_PER_W + jnp.where(wid < _EXTRA, 1, 0)

    def _chunk(i, _):
        row0 = base_row + i * _CH
        pltpu.sync_copy(col_hbm.at[pl.ds(row0, _CH)], idx_buf)
        pltpu.sync_copy(ea_hbm.at[pl.ds(row0 * _LANE, _CH_E)], rows_buf)
        for j in range(_CH):
            pltpu.sync_copy(rows_buf.at[pl.ds(j * _LANE, _LANE)],
                            acc.at[idx_buf.at[j]], add=True)
        return 0

    lax.fori_loop(0, nchunks, _chunk, 0)
    plsc.subcore_barrier()

    @pl.when(cid == 0)
    def _():
        pltpu.sync_copy(acc.at[pl.ds(base_n, _NPT)],
                        out0_hbm.at[pl.ds(base_n, _NPT)])

    @pl.when(cid == 1)
    def _():
        pltpu.sync_copy(acc.at[pl.ds(base_n, _NPT)],
                        out1_hbm.at[pl.ds(base_n, _NPT)])


_seg_sum = pl.kernel(
    _seg_sum_body,
    out_type=(jax.ShapeDtypeStruct((_NPAD, _D_EDGE), jnp.float32),
              jax.ShapeDtypeStruct((_NPAD, _D_EDGE), jnp.float32)),
    mesh=plsc.VectorSubcoreMesh(core_axis_name="c", subcore_axis_name="s"),
    compiler_params=pltpu.CompilerParams(use_tc_tiling_on_sc=False),
    scratch_types=[
        pltpu.VMEM((_CH, _LANE), jnp.int32),
        pltpu.VMEM((_CH_E, _D_EDGE), jnp.float32),
        pltpu.VMEM_SHARED((_NPAD, _D_EDGE), jnp.float32),
    ],
)


_BN = 2000


def _mlp_body(x_ref, p0_ref, p1_ref, g_ref, w1x_ref, w1a_ref, w1g_ref,
              b1_ref, w2_ref, b2_ref, w3_ref, b3_ref, w4_ref, b4_ref,
              out_ref):
    agg = p0_ref[...] + p1_ref[...]
    c = (jnp.dot(g_ref[...], w1g_ref[...], preferred_element_type=jnp.float32)
         + b1_ref[...])
    h = (jnp.dot(x_ref[...], w1x_ref[...], preferred_element_type=jnp.float32)
         + jnp.dot(agg, w1a_ref[...], preferred_element_type=jnp.float32)
         + c)
    h = jnp.maximum(h, 0.0)
    h = jnp.dot(h, w2_ref[...], preferred_element_type=jnp.float32) + b2_ref[...]
    h = jnp.maximum(h, 0.0)
    h = jnp.dot(h, w3_ref[...], preferred_element_type=jnp.float32) + b3_ref[...]
    h = jnp.maximum(h, 0.0)
    out_ref[...] = (jnp.dot(h, w4_ref[...], preferred_element_type=jnp.float32)
                    + b4_ref[...])


def _row_block(i):
    return (i, 0)


def _whole(i):
    return (0, 0)


@functools.partial(jax.jit, static_argnums=())
def kernel(x, edge_index, edge_attr, global_attr,
           W1, b1, W2, b2, W3, b3, W4, b4):
    col2d = _extract_col(edge_index)
    p0, p1 = _seg_sum(col2d, edge_attr)

    d_node = x.shape[1]
    w1x = W1[:, :d_node].T
    w1a = W1[:, d_node:d_node + _D_EDGE].T
    w1g = W1[:, d_node + _D_EDGE:].T

    grid = (_N // _BN,)
    out = pl.pallas_call(
        _mlp_body,
        grid=grid,
        in_specs=[
            pl.BlockSpec((_BN, d_node), _row_block),
            pl.BlockSpec((_BN, _D_EDGE), _row_block),
            pl.BlockSpec((_BN, _D_EDGE), _row_block),
            pl.BlockSpec(global_attr.shape, _whole),
            pl.BlockSpec(w1x.shape, _whole),
            pl.BlockSpec(w1a.shape, _whole),
            pl.BlockSpec(w1g.shape, _whole),
            pl.BlockSpec((1, b1.shape[0]), _whole),
            pl.BlockSpec(W2.shape, _whole),
            pl.BlockSpec((1, b2.shape[0]), _whole),
            pl.BlockSpec(W3.shape, _whole),
            pl.BlockSpec((1, b3.shape[0]), _whole),
            pl.BlockSpec(W4.shape, _whole),
            pl.BlockSpec((1, b4.shape[0]), _whole),
        ],
        out_specs=pl.BlockSpec((_BN, W4.shape[0]), _row_block),
        out_shape=jax.ShapeDtypeStruct((_N, W4.shape[0]), jnp.float32),
    )(x, p0, p1, global_attr, w1x, w1a, w1g, b1.reshape(1, -1),
      W2.T, b2.reshape(1, -1), W3.T, b3.reshape(1, -1), W4.T,
      b4.reshape(1, -1))
    return out
